# baseline (speedup 1.0000x reference)
import functools

import jax
import jax.numpy as jnp
from jax import lax
from jax.experimental import pallas as pl
from jax.experimental.pallas import tpu as pltpu
from jax.experimental.pallas import tpu_sc as plsc

_NC = 2
_NS = 16
_NW = _NC * _NS
_CHUNK = 128


def _gather_body(n_chunks, emb_dim, idx_hbm, table_hbm, out_hbm,
                 idx_v, rows_v, sem):
  wid = lax.axis_index("s") * _NC + lax.axis_index("c")
  pltpu.sync_copy(idx_hbm.at[wid], idx_v)
  base = wid * (n_chunks * _CHUNK)

  def body(j, carry):
    pltpu.async_copy(table_hbm.at[idx_v.at[j]], rows_v, sem).wait()
    pltpu.sync_copy(rows_v, out_hbm.at[pl.ds(base + j * _CHUNK, _CHUNK)])
    return carry

  lax.fori_loop(0, n_chunks, body, 0, unroll=False)


@functools.partial(jax.jit, static_argnames=("n_chunks", "emb_dim"))
def _sc_gather(idx, table, *, n_chunks, emb_dim):
  mesh = plsc.VectorSubcoreMesh(
      core_axis_name="c", subcore_axis_name="s",
      num_cores=_NC, num_subcores=_NS)
  total = _NW * n_chunks * _CHUNK
  run = pl.kernel(
      functools.partial(_gather_body, n_chunks, emb_dim),
      out_type=jax.ShapeDtypeStruct((total, emb_dim), jnp.float32),
      mesh=mesh,
      scratch_types=[
          pltpu.VMEM((n_chunks, _CHUNK), jnp.int32),
          pltpu.VMEM((_CHUNK, emb_dim), jnp.float32),
          pltpu.SemaphoreType.DMA,
      ],
      compiler_params=pltpu.CompilerParams(use_tc_tiling_on_sc=False),
  )
  return run(idx, table)


def kernel(indices, table):
  bsz, seq = indices.shape
  vocab, emb_dim = table.shape
  flat = indices.reshape(-1).astype(jnp.int32)
  total = bsz * seq
  n_chunks = total // (_NW * _CHUNK)
  assert n_chunks * _NW * _CHUNK == total
  idx3 = flat.reshape(_NW, n_chunks, _CHUNK)
  out = _sc_gather(idx3, table, n_chunks=n_chunks, emb_dim=emb_dim)
  return out.reshape(bsz, seq, emb_dim)


# speedup vs baseline: 1.0450x; 1.0450x over previous
import functools

import jax
import jax.numpy as jnp
from jax import lax
from jax.experimental import pallas as pl
from jax.experimental.pallas import tpu as pltpu
from jax.experimental.pallas import tpu_sc as plsc

_NC = 2
_NS = 16
_NW = _NC * _NS
_CHUNK = 128
_NBUF = 5


def _gather_body(n_chunks, emb_dim, idx_hbm, table_hbm, out_hbm,
                 idx_v, rows_v, gsems, osems):
  wid = lax.axis_index("s") * _NC + lax.axis_index("c")
  pltpu.sync_copy(idx_hbm.at[wid], idx_v)
  base = wid * (n_chunks * _CHUNK)

  def gather_copy(b, c):
    return pltpu.make_async_copy(table_hbm.at[idx_v.at[c]], rows_v.at[b],
                                 gsems[b])

  def out_copy(b, c):
    return pltpu.make_async_copy(
        rows_v.at[b], out_hbm.at[pl.ds(base + c * _CHUNK, _CHUNK)], osems[b])

  for b in range(_NBUF):
    gather_copy(b, b).start()

  n_steady = n_chunks // _NBUF - 1

  def body(i, carry):
    k = i * _NBUF
    for b in range(_NBUF):
      gather_copy(b, k + b).wait()
      out_copy(b, k + b).start()
    for b in range(_NBUF):
      out_copy(b, k + b).wait()
      gather_copy(b, k + b + _NBUF).start()
    return carry

  lax.fori_loop(0, n_steady, body, 0, unroll=False)

  k = n_steady * _NBUF
  for b in range(_NBUF):
    gather_copy(b, k + b).wait()
    out_copy(b, k + b).start()
  for b in range(_NBUF):
    out_copy(b, k + b).wait()


@functools.partial(jax.jit, static_argnames=("n_chunks", "emb_dim"))
def _sc_gather(idx, table, *, n_chunks, emb_dim):
  mesh = plsc.VectorSubcoreMesh(
      core_axis_name="c", subcore_axis_name="s",
      num_cores=_NC, num_subcores=_NS)
  total = _NW * n_chunks * _CHUNK
  run = pl.kernel(
      functools.partial(_gather_body, n_chunks, emb_dim),
      out_type=jax.ShapeDtypeStruct((total, emb_dim), jnp.float32),
      mesh=mesh,
      scratch_types=[
          pltpu.VMEM((n_chunks, _CHUNK), jnp.int32),
          pltpu.VMEM((_NBUF, _CHUNK, emb_dim), jnp.float32),
          [pltpu.SemaphoreType.DMA] * _NBUF,
          [pltpu.SemaphoreType.DMA] * _NBUF,
      ],
      compiler_params=pltpu.CompilerParams(use_tc_tiling_on_sc=False),
  )
  return run(idx, table)


def kernel(indices, table):
  bsz, seq = indices.shape
  vocab, emb_dim = table.shape
  flat = indices.reshape(-1).astype(jnp.int32)
  total = bsz * seq
  n_chunks = total // (_NW * _CHUNK)
  assert n_chunks * _NW * _CHUNK == total and n_chunks % _NBUF == 0
  idx3 = flat.reshape(_NW, n_chunks, _CHUNK)
  out = _sc_gather(idx3, table, n_chunks=n_chunks, emb_dim=emb_dim)
  return out.reshape(bsz, seq, emb_dim)


# speedup vs baseline: 1.3145x; 1.2579x over previous
import functools

import jax
import jax.numpy as jnp
from jax import lax
from jax.experimental import pallas as pl
from jax.experimental.pallas import tpu as pltpu
from jax.experimental.pallas import tpu_sc as plsc

_NC = 2
_NS = 16
_NW = _NC * _NS
_CHUNK = 128
_NBUF = 5


def _gather_body(n_blocks, seq, bsz, emb_dim, idx_hbm, table_hbm, out_hbm,
                 idx_v, rows_v, isem, gsems, osems):
  wid = lax.axis_index("s") * _NC + lax.axis_index("c")
  nb_per_l = bsz // _CHUNK
  b0 = wid * n_blocks

  def stage(j, carry):
    bid = b0 + j
    l = bid // nb_per_l
    c = bid % nb_per_l
    pltpu.make_async_copy(
        idx_hbm.at[l, pl.ds(c * _CHUNK, _CHUNK)], idx_v.at[j], isem).start()
    return carry

  lax.fori_loop(0, n_blocks, stage, 0, unroll=False)

  def drain(j, carry):
    pltpu.make_async_copy(
        idx_hbm.at[0, pl.ds(0, _CHUNK)], idx_v.at[j], isem).wait()
    return carry

  lax.fori_loop(0, n_blocks, drain, 0, unroll=False)

  def gather_copy(b, j):
    return pltpu.make_async_copy(table_hbm.at[idx_v.at[j]], rows_v.at[b],
                                 gsems[b])

  def out_copy(b, j):
    bid = b0 + j
    l = bid // nb_per_l
    c = bid % nb_per_l
    return pltpu.make_async_copy(
        rows_v.at[b], out_hbm.at[l, pl.ds(c * _CHUNK, _CHUNK)], osems[b])

  for b in range(_NBUF):
    gather_copy(b, b).start()

  n_steady = n_blocks // _NBUF - 1

  def body(i, carry):
    k = i * _NBUF
    for b in range(_NBUF):
      gather_copy(b, k + b).wait()
      out_copy(b, k + b).start()
    for b in range(_NBUF):
      out_copy(b, k + b).wait()
      gather_copy(b, k + b + _NBUF).start()
    return carry

  lax.fori_loop(0, n_steady, body, 0, unroll=False)

  k = n_steady * _NBUF
  for b in range(_NBUF):
    gather_copy(b, k + b).wait()
    out_copy(b, k + b).start()
  for b in range(_NBUF):
    out_copy(b, k + b).wait()


@functools.partial(jax.jit, static_argnames=("seq", "bsz", "emb_dim"))
def _sc_gather(idx, table, *, seq, bsz, emb_dim):
  mesh = plsc.VectorSubcoreMesh(
      core_axis_name="c", subcore_axis_name="s",
      num_cores=_NC, num_subcores=_NS)
  n_blocks = seq * bsz // (_NW * _CHUNK)
  run = pl.kernel(
      functools.partial(_gather_body, n_blocks, seq, bsz, emb_dim),
      out_type=jax.ShapeDtypeStruct((seq, bsz, emb_dim), jnp.float32),
      mesh=mesh,
      scratch_types=[
          pltpu.VMEM((n_blocks, _CHUNK), jnp.int32),
          pltpu.VMEM((_NBUF, _CHUNK, emb_dim), jnp.float32),
          pltpu.SemaphoreType.DMA,
          [pltpu.SemaphoreType.DMA] * _NBUF,
          [pltpu.SemaphoreType.DMA] * _NBUF,
      ],
      compiler_params=pltpu.CompilerParams(use_tc_tiling_on_sc=False),
  )
  return run(idx, table)


def kernel(indices, table):
  bsz, seq = indices.shape
  vocab, emb_dim = table.shape
  idx_t = indices.T.astype(jnp.int32)
  out_t = _sc_gather(idx_t, table, seq=seq, bsz=bsz, emb_dim=emb_dim)
  return out_t.transpose(1, 0, 2)
